# Initial kernel scaffold; baseline (speedup 1.0000x reference)
import jax
import jax.numpy as jnp
from jax.experimental import pallas as pl


def kernel(inputs, embedding):
    raise NotImplementedError("write your pallas kernel here")


# speedup vs baseline: 1.2317x; 1.2317x over previous
import functools

import jax
import jax.numpy as jnp
from jax.experimental import pallas as pl
from jax.experimental.pallas import tpu as pltpu
from jax.experimental.pallas import tpu_sc as plsc

_NUM_E = 8192
_DIM = 64
_N = 16384
_BLK = 256
_NBLK = _N // _BLK
_GATHER_WIN = 128
_COMMIT = 0.25


def _dist_body(x_ref, et_ref, en_ref, idx_ref, loss_ref):
    x = jnp.clip(x_ref[...], -1.0, 1.0)
    xn = jnp.sum(x * x, axis=1, keepdims=True)
    mm = jax.lax.dot_general(
        x.astype(jnp.bfloat16), et_ref[...].astype(jnp.bfloat16),
        (((1,), (0,)), ((), ())),
        preferred_element_type=jnp.float32)
    scores = (xn + en_ref[...]) - 2.0 * mm
    half = _NUM_E // 2
    s1 = scores[:, :half]
    s2 = scores[:, half:]
    v1 = jnp.min(s1, axis=1)
    i1 = jnp.argmin(s1, axis=1).astype(jnp.int32)
    v2 = jnp.min(s2, axis=1)
    i2 = jnp.argmin(s2, axis=1).astype(jnp.int32)
    v1r = v1.astype(jnp.bfloat16).astype(jnp.float32)
    win2 = v2 < v1r
    idx_ref[0, 0, :] = jnp.where(win2, i2 + half, i1)
    m = jnp.where(win2, v2, v1)

    @pl.when(pl.program_id(0) == 0)
    def _init():
        loss_ref[...] = jnp.zeros((1, 1), jnp.float32)

    loss_ref[...] += jnp.sum(m).reshape(1, 1)


def _distances_argmin(x_flat, et, en):
    return pl.pallas_call(
        _dist_body,
        grid=(_NBLK,),
        in_specs=[
            pl.BlockSpec((_BLK, _DIM), lambda i: (i, 0)),
            pl.BlockSpec((_DIM, _NUM_E), lambda i: (0, 0)),
            pl.BlockSpec((1, _NUM_E), lambda i: (0, 0)),
        ],
        out_specs=[
            pl.BlockSpec((1, 1, _BLK), lambda i: (i, 0, 0)),
            pl.BlockSpec((1, 1), lambda i: (0, 0)),
        ],
        out_shape=[
            jax.ShapeDtypeStruct((_NBLK, 1, _BLK), jnp.int32),
            jax.ShapeDtypeStruct((1, 1), jnp.float32),
        ],
    )(x_flat, et, en)


_SC_CORES = 2
_SC_SUBCORES = 16
_SC_WORKERS = _SC_CORES * _SC_SUBCORES
_BPW = _N // _SC_WORKERS
_CHUNK = 128
_NCHUNK = _BPW // _CHUNK
_TW = 128


def _sc_gather(emb, idx_flat):
    mesh = plsc.VectorSubcoreMesh(core_axis_name="c", subcore_axis_name="s")

    @functools.partial(
        pl.kernel, mesh=mesh,
        out_type=jax.ShapeDtypeStruct((_N, _TW), emb.dtype),
        scratch_types=[
            pltpu.VMEM((_CHUNK,), jnp.int32),
            pltpu.VMEM((_CHUNK, _TW), jnp.float32),
            pltpu.SemaphoreType.DMA,
        ],
    )
    def k(table_hbm, idx_hbm, out_hbm, idx_v, rows_v, sem):
        wid = jax.lax.axis_index("s") * _SC_CORES + jax.lax.axis_index("c")
        base = wid * _BPW

        @pl.loop(0, _NCHUNK)
        def _(c):
            off = base + c * _CHUNK
            pltpu.sync_copy(idx_hbm.at[pl.ds(off, _CHUNK)], idx_v)
            pltpu.async_copy(table_hbm.at[idx_v], rows_v, sem).wait()
            pltpu.sync_copy(rows_v, out_hbm.at[pl.ds(off, _CHUNK)])

    return k(emb, idx_flat)


def kernel(inputs, embedding):
    x_flat = inputs.reshape(_N, _DIM)
    et = embedding.T
    en = jnp.sum(embedding * embedding, axis=1)[None, :]
    idx3, loss_acc = _distances_argmin(x_flat, et, en)
    idx_flat = idx3.reshape(_N)
    table = jnp.pad(embedding, ((0, 0), (0, _TW - _DIM)))
    gathered = _sc_gather(table, idx_flat)
    quantized = gathered[:, :_DIM].reshape(inputs.shape)
    loss = (1.0 + _COMMIT) * loss_acc[0, 0] / jnp.float32(_N * _DIM)
    return (quantized, loss)


# speedup vs baseline: 1.4511x; 1.1781x over previous
import functools

import jax
import jax.numpy as jnp
from jax.experimental import pallas as pl
from jax.experimental.pallas import tpu as pltpu
from jax.experimental.pallas import tpu_sc as plsc

_NUM_E = 8192
_DIM = 64
_N = 16384
_BLK = 256
_NBLK = _N // _BLK
_GATHER_WIN = 128
_COMMIT = 0.25


_LANES = 128


def _half_tournament(xn, en, mm2, lo_blk, n_blk):
    iota = jax.lax.broadcasted_iota(jnp.int32, (_BLK, _LANES), 1)
    big = jnp.int32(1 << 30)
    acc_v = None
    for b in range(n_blk):
        col = (lo_blk + b) * _LANES
        t = xn + en[:, col:col + _LANES]
        cand_v = t + mm2[:, col:col + _LANES]
        if acc_v is None:
            acc_v, acc_i = cand_v, iota + col
        else:
            win = cand_v < acc_v
            acc_v = jnp.where(win, cand_v, acc_v)
            acc_i = jnp.where(win, iota + col, acc_i)
    row_v = jnp.min(acc_v, axis=1)
    tied = acc_v == row_v[:, None]
    row_i = jnp.min(jnp.where(tied, acc_i, big), axis=1)
    return row_v, row_i


def _dist_body(x_ref, et_ref, en_ref, idx_ref, loss_ref):
    x = jnp.clip(x_ref[...], -1.0, 1.0)
    xn = jnp.sum(x * x, axis=1, keepdims=True)
    mm2 = jax.lax.dot_general(
        (-2.0 * x).astype(jnp.bfloat16), et_ref[...].astype(jnp.bfloat16),
        (((1,), (0,)), ((), ())),
        preferred_element_type=jnp.float32)
    half = _NUM_E // 2
    nb = half // _LANES
    en = en_ref[...]
    v1, i1 = _half_tournament(xn, en, mm2, 0, nb)
    v2, i2 = _half_tournament(xn, en, mm2, nb, nb)
    v1r = v1.astype(jnp.bfloat16).astype(jnp.float32)
    win2 = v2 < v1r
    idx_ref[0, 0, :] = jnp.where(win2, i2, i1)
    m = jnp.where(win2, v2, v1)

    @pl.when(pl.program_id(0) == 0)
    def _init():
        loss_ref[...] = jnp.zeros((1, 1), jnp.float32)

    loss_ref[...] += jnp.sum(m).reshape(1, 1)


def _distances_argmin(x_flat, et, en):
    return pl.pallas_call(
        _dist_body,
        grid=(_NBLK,),
        in_specs=[
            pl.BlockSpec((_BLK, _DIM), lambda i: (i, 0)),
            pl.BlockSpec((_DIM, _NUM_E), lambda i: (0, 0)),
            pl.BlockSpec((1, _NUM_E), lambda i: (0, 0)),
        ],
        out_specs=[
            pl.BlockSpec((1, 1, _BLK), lambda i: (i, 0, 0)),
            pl.BlockSpec((1, 1), lambda i: (0, 0)),
        ],
        out_shape=[
            jax.ShapeDtypeStruct((_NBLK, 1, _BLK), jnp.int32),
            jax.ShapeDtypeStruct((1, 1), jnp.float32),
        ],
    )(x_flat, et, en)


_SC_CORES = 2
_SC_SUBCORES = 16
_SC_WORKERS = _SC_CORES * _SC_SUBCORES
_BPW = _N // _SC_WORKERS
_CHUNK = 128
_NCHUNK = _BPW // _CHUNK
_TW = 128


def _sc_gather(emb, idx_flat):
    mesh = plsc.VectorSubcoreMesh(core_axis_name="c", subcore_axis_name="s")

    @functools.partial(
        pl.kernel, mesh=mesh,
        out_type=jax.ShapeDtypeStruct((_N, _TW), emb.dtype),
        scratch_types=[
            pltpu.VMEM((_CHUNK,), jnp.int32),
            pltpu.VMEM((_CHUNK, _TW), jnp.float32),
            pltpu.SemaphoreType.DMA,
        ],
    )
    def k(table_hbm, idx_hbm, out_hbm, idx_v, rows_v, sem):
        wid = jax.lax.axis_index("s") * _SC_CORES + jax.lax.axis_index("c")
        base = wid * _BPW

        @pl.loop(0, _NCHUNK)
        def _(c):
            off = base + c * _CHUNK
            pltpu.sync_copy(idx_hbm.at[pl.ds(off, _CHUNK)], idx_v)
            pltpu.async_copy(table_hbm.at[idx_v], rows_v, sem).wait()
            pltpu.sync_copy(rows_v, out_hbm.at[pl.ds(off, _CHUNK)])

    return k(emb, idx_flat)


def kernel(inputs, embedding):
    x_flat = inputs.reshape(_N, _DIM)
    et = embedding.T
    en = jnp.sum(embedding * embedding, axis=1)[None, :]
    idx3, loss_acc = _distances_argmin(x_flat, et, en)
    idx_flat = idx3.reshape(_N)
    table = jnp.pad(embedding, ((0, 0), (0, _TW - _DIM)))
    gathered = _sc_gather(table, idx_flat)
    quantized = gathered[:, :_DIM].reshape(inputs.shape)
    loss = (1.0 + _COMMIT) * loss_acc[0, 0] / jnp.float32(_N * _DIM)
    return (quantized, loss)
